# baseline (speedup 1.0000x reference)
import jax
import jax.numpy as jnp
from jax import lax
from jax.experimental import pallas as pl
from jax.experimental.pallas import tpu as pltpu
from jax.experimental.pallas import tpu_sc as plsc

_NUM_CLASSES = 100000
_FEAT = 64
_BATCH = 16384
_NC = 2
_NS = 16
_NW = _NC * _NS
_BPW = _BATCH // _NW
_CHUNK = 128
_NCHUNK = _BPW // _CHUNK


def _cl_kernel(feat_hbm, lab_hbm, cent_hbm, out_hbm,
               lab_v, rows_v, feat_v, acc_v, sem):
    wid = lax.axis_index("c") * _NS + lax.axis_index("s")
    base = wid * _BPW

    pltpu.sync_copy(lab_hbm.at[pl.ds(wid * _NCHUNK, _NCHUNK)], lab_v)

    zeros = jnp.zeros((16,), jnp.float32)
    accs = (zeros, zeros, zeros, zeros)
    for j in range(_NCHUNK):
        gat = pltpu.async_copy(cent_hbm.at[lab_v.at[j]], rows_v, sem)
        pltpu.sync_copy(feat_hbm.at[pl.ds(base + j * _CHUNK, _CHUNK)], feat_v)
        gat.wait()

        def row_body(r, accs, _rows=rows_v, _feat=feat_v):
            a0, a1, a2, a3 = accs
            f0 = _feat[r, pl.ds(0, 16)]
            c0 = _rows[r, pl.ds(0, 16)]
            d0 = f0 - c0
            a0 = a0 + d0 * d0
            f1 = _feat[r, pl.ds(16, 16)]
            c1 = _rows[r, pl.ds(16, 16)]
            d1 = f1 - c1
            a1 = a1 + d1 * d1
            f2 = _feat[r, pl.ds(32, 16)]
            c2 = _rows[r, pl.ds(32, 16)]
            d2 = f2 - c2
            a2 = a2 + d2 * d2
            f3 = _feat[r, pl.ds(48, 16)]
            c3 = _rows[r, pl.ds(48, 16)]
            d3 = f3 - c3
            a3 = a3 + d3 * d3
            return (a0, a1, a2, a3)

        accs = lax.fori_loop(0, _CHUNK, row_body, accs)

    total = (accs[0] + accs[1]) + (accs[2] + accs[3])
    acc_v[...] = total * jnp.float32(1.0 / _BATCH)
    pltpu.sync_copy(acc_v, out_hbm.at[wid])


@jax.jit
def _center_loss(features, labels, centers):
    labels2 = labels.reshape(_BATCH // _CHUNK, _CHUNK)
    mesh = plsc.VectorSubcoreMesh(
        core_axis_name="c", subcore_axis_name="s",
        num_cores=_NC, num_subcores=_NS)
    out = pl.kernel(
        _cl_kernel,
        out_type=jax.ShapeDtypeStruct((_NW, 16), jnp.float32),
        mesh=mesh,
        compiler_params=pltpu.CompilerParams(use_tc_tiling_on_sc=False),
        scratch_types=[
            pltpu.VMEM((_NCHUNK, _CHUNK), jnp.int32),
            pltpu.VMEM((_CHUNK, _FEAT), jnp.float32),
            pltpu.VMEM((_CHUNK, _FEAT), jnp.float32),
            pltpu.VMEM((16,), jnp.float32),
            pltpu.SemaphoreType.DMA,
        ],
    )(features, labels2, centers)
    return jnp.sum(out)


def kernel(features, labels, centers):
    return _center_loss(features, labels.astype(jnp.int32), centers)
